# Initial kernel scaffold; baseline (speedup 1.0000x reference)
import jax
import jax.numpy as jnp
from jax.experimental import pallas as pl
from jax.experimental.pallas import tpu as pltpu

T = 2048
DIN = 2048
D = 1024
F = 4096
E = 8
K = 2
B = 256
R = (T * K) // B + E
N = R * B
TB = 256
FB = 1024
NF = F // FB
EPAD = 128


def _gate_body(x_ref, wg1_ref, bg1_ref, wg2_ref, bg2_ref, meta_ref):
    x = x_ref[...]
    h = jax.nn.gelu(
        jnp.dot(x, wg1_ref[...], preferred_element_type=jnp.float32)
        + bg1_ref[...], approximate=False)
    logits = jnp.dot(h, wg2_ref[...], preferred_element_type=jnp.float32) + bg2_ref[...]
    col = jax.lax.broadcasted_iota(jnp.int32, (TB, EPAD), 1)
    valid = col < E
    lm = jnp.where(valid, logits, jnp.float32(-1e30))
    m = jnp.max(lm, axis=1, keepdims=True)
    p = jnp.where(valid, jnp.exp(lm - m), 0.0)
    s = jnp.sum(p, axis=1, keepdims=True)
    probs = p / s
    m1 = jnp.max(probs, axis=1, keepdims=True)
    i1 = jnp.min(jnp.where(probs == m1, col, EPAD - 1), axis=1, keepdims=True)
    probs2 = jnp.where(col == i1, -1.0, probs)
    m2 = jnp.max(probs2, axis=1, keepdims=True)
    i2 = jnp.min(jnp.where(probs2 == m2, col, EPAD - 1), axis=1, keepdims=True)
    tot = m1 + m2
    w1 = m1 / tot
    w2 = m2 / tot
    col8 = jax.lax.broadcasted_iota(jnp.int32, (TB, 8), 1)
    out = (jnp.where(col8 == 0, i1.astype(jnp.float32), 0.0)
           + jnp.where(col8 == 1, i2.astype(jnp.float32), 0.0)
           + jnp.where(col8 == 2, w1, 0.0)
           + jnp.where(col8 == 3, w2, 0.0))
    meta_ref[...] = out


def _gather_body(rt_ref, x_ref, out_ref):
    rt = rt_ref[0]
    colt = jax.lax.broadcasted_iota(jnp.int32, (B, T), 1)
    p = (colt == rt).astype(jnp.float32)
    out_ref[...] = jnp.dot(p, x_ref[...], preferred_element_type=jnp.float32)


def _mm1_body(te_ref, tv_ref, xs_ref, w1_ref, b1_ref, h_ref):
    r = pl.program_id(1)

    @pl.when(tv_ref[r] == 1)
    def _():
        h_ref[...] = jax.nn.gelu(
            jnp.dot(xs_ref[...], w1_ref[0], preferred_element_type=jnp.float32)
            + b1_ref[0], approximate=False)

    @pl.when(tv_ref[r] == 0)
    def _():
        h_ref[...] = jnp.zeros_like(h_ref)


def _mm2_body(te_ref, tv_ref, h_ref, w2_ref, b2_ref, y_ref):
    r = pl.program_id(0)

    @pl.when(tv_ref[r] == 1)
    def _():
        y_ref[...] = jax.nn.sigmoid(
            jnp.dot(h_ref[...], w2_ref[0], preferred_element_type=jnp.float32)
            + b2_ref[0])

    @pl.when(tv_ref[r] == 0)
    def _():
        y_ref[...] = jnp.zeros_like(y_ref)


def _combine_body(p0_ref, p1_ref, w0_ref, w1_ref, y_ref, out_ref):
    coln = jax.lax.broadcasted_iota(jnp.int32, (TB, N), 1)
    s = (jnp.where(coln == p0_ref[...], w0_ref[...], 0.0)
         + jnp.where(coln == p1_ref[...], w1_ref[...], 0.0))
    out_ref[...] = jnp.dot(s, y_ref[...], preferred_element_type=jnp.float32)


def kernel(combined, Wg1, bg1, Wg2, bg2, W1, b1, W2, b2):
    f32 = jnp.float32
    wg2p = jnp.pad(Wg2, ((0, 0), (0, EPAD - E)))
    bg2p = jnp.pad(bg2, (0, EPAD - E)).reshape(1, EPAD)
    bg1r = bg1.reshape(1, D)
    meta = pl.pallas_call(
        _gate_body,
        grid=(T // TB,),
        in_specs=[
            pl.BlockSpec((TB, DIN), lambda i: (i, 0)),
            pl.BlockSpec((DIN, D), lambda i: (0, 0)),
            pl.BlockSpec((1, D), lambda i: (0, 0)),
            pl.BlockSpec((D, EPAD), lambda i: (0, 0)),
            pl.BlockSpec((1, EPAD), lambda i: (0, 0)),
        ],
        out_specs=pl.BlockSpec((TB, 8), lambda i: (i, 0)),
        out_shape=jax.ShapeDtypeStruct((T, 8), f32),
    )(combined, Wg1, bg1r, wg2p, bg2p)

    e0 = meta[:, 0].astype(jnp.int32)
    e1 = meta[:, 1].astype(jnp.int32)
    w0 = meta[:, 2:3]
    w1v = meta[:, 3:4]

    e_all = jnp.stack([e0, e1], axis=1).reshape(-1)
    onehot = (e_all[:, None] == jnp.arange(E, dtype=jnp.int32)[None, :]).astype(jnp.int32)
    ranks = jnp.cumsum(onehot, axis=0)
    counts = ranks[-1]
    rank_p = jnp.take_along_axis(ranks, e_all[:, None], axis=1)[:, 0] - 1
    padded = ((counts + B - 1) // B) * B
    poff = jnp.concatenate([jnp.zeros(1, jnp.int32),
                            jnp.cumsum(padded)[:-1].astype(jnp.int32)])
    dst = poff[e_all] + rank_p
    tok = jnp.arange(T * K, dtype=jnp.int32) // K
    row_token = jnp.zeros(N, jnp.int32).at[dst].set(tok)
    pos = dst.reshape(T, K)
    tile_start = jnp.arange(R, dtype=jnp.int32) * B
    te = jnp.clip(jnp.searchsorted(poff, tile_start, side='right') - 1,
                  0, E - 1).astype(jnp.int32)
    tv = (tile_start < poff[te] + counts[te]).astype(jnp.int32)

    xs = pl.pallas_call(
        _gather_body,
        grid=(R,),
        in_specs=[
            pl.BlockSpec((1, B, 1), lambda r: (r, 0, 0)),
            pl.BlockSpec((T, DIN), lambda r: (0, 0)),
        ],
        out_specs=pl.BlockSpec((B, DIN), lambda r: (r, 0)),
        out_shape=jax.ShapeDtypeStruct((N, DIN), f32),
    )(row_token.reshape(R, B, 1), combined)

    b1r = b1.reshape(E * NF, 1, FB)
    h = pl.pallas_call(
        _mm1_body,
        grid_spec=pltpu.PrefetchScalarGridSpec(
            num_scalar_prefetch=2,
            grid=(NF, R),
            in_specs=[
                pl.BlockSpec((B, DIN), lambda f, r, te, tv: (r, 0)),
                pl.BlockSpec((1, DIN, FB), lambda f, r, te, tv: (te[r], 0, f)),
                pl.BlockSpec((1, 1, FB), lambda f, r, te, tv: (te[r] * NF + f, 0, 0)),
            ],
            out_specs=pl.BlockSpec((B, FB), lambda f, r, te, tv: (r, f)),
        ),
        out_shape=jax.ShapeDtypeStruct((N, F), f32),
    )(te, tv, xs, W1, b1r)

    b2r = b2.reshape(E, 1, D)
    y = pl.pallas_call(
        _mm2_body,
        grid_spec=pltpu.PrefetchScalarGridSpec(
            num_scalar_prefetch=2,
            grid=(R,),
            in_specs=[
                pl.BlockSpec((B, F), lambda r, te, tv: (r, 0)),
                pl.BlockSpec((1, F, D), lambda r, te, tv: (te[r], 0, 0)),
                pl.BlockSpec((1, 1, D), lambda r, te, tv: (te[r], 0, 0)),
            ],
            out_specs=pl.BlockSpec((B, D), lambda r, te, tv: (r, 0)),
        ),
        out_shape=jax.ShapeDtypeStruct((N, D), f32),
    )(te, tv, h, W2, b2r)

    fused = pl.pallas_call(
        _combine_body,
        grid=(T // TB,),
        in_specs=[
            pl.BlockSpec((TB, 1), lambda i: (i, 0)),
            pl.BlockSpec((TB, 1), lambda i: (i, 0)),
            pl.BlockSpec((TB, 1), lambda i: (i, 0)),
            pl.BlockSpec((TB, 1), lambda i: (i, 0)),
            pl.BlockSpec((N, D), lambda i: (0, 0)),
        ],
        out_specs=pl.BlockSpec((TB, D), lambda i: (i, 0)),
        out_shape=jax.ShapeDtypeStruct((T, D), f32),
    )(pos[:, 0:1], pos[:, 1:2], w0, w1v, y)
    return fused


# speedup vs baseline: 2.6293x; 2.6293x over previous
import jax
import jax.numpy as jnp
from jax.experimental import pallas as pl
from jax.experimental.pallas import tpu as pltpu

T = 2048
DIN = 2048
D = 1024
F = 4096
E = 8
K = 2
B = 256
R = (T * K) // B + E
N = R * B
TB = 256
FB = 1024
NF = F // FB
EPAD = 128

_INV_SQRT2 = 0.7071067811865476


def _gelu(x):
    return 0.5 * x * (1.0 + jax.lax.erf(x * _INV_SQRT2))


def _gate_body(x_ref, wg1_ref, bg1_ref, wg2_ref, bg2_ref, meta_ref):
    x = x_ref[...]
    h = _gelu(jnp.dot(x, wg1_ref[...], preferred_element_type=jnp.float32)
              + bg1_ref[...])
    logits = jnp.dot(h, wg2_ref[...], preferred_element_type=jnp.float32) + bg2_ref[...]
    col = jax.lax.broadcasted_iota(jnp.int32, (TB, EPAD), 1)
    valid = col < E
    lm = jnp.where(valid, logits, jnp.float32(-1e30))
    m = jnp.max(lm, axis=1, keepdims=True)
    p = jnp.where(valid, jnp.exp(lm - m), 0.0)
    s = jnp.sum(p, axis=1, keepdims=True)
    probs = p / s
    m1 = jnp.max(probs, axis=1, keepdims=True)
    i1 = jnp.min(jnp.where(probs == m1, col, EPAD - 1), axis=1, keepdims=True)
    probs2 = jnp.where(col == i1, -1.0, probs)
    m2 = jnp.max(probs2, axis=1, keepdims=True)
    i2 = jnp.min(jnp.where(probs2 == m2, col, EPAD - 1), axis=1, keepdims=True)
    tot = m1 + m2
    w1 = m1 / tot
    w2 = m2 / tot
    col8 = jax.lax.broadcasted_iota(jnp.int32, (TB, 8), 1)
    out = (jnp.where(col8 == 0, i1.astype(jnp.float32), 0.0)
           + jnp.where(col8 == 1, i2.astype(jnp.float32), 0.0)
           + jnp.where(col8 == 2, w1, 0.0)
           + jnp.where(col8 == 3, w2, 0.0))
    meta_ref[...] = out


def _gather_body(rt_ref, x_ref, out_ref):
    rt = rt_ref[0]
    colt = jax.lax.broadcasted_iota(jnp.int32, (B, T), 1)
    p = (colt == rt).astype(jnp.float32)
    out_ref[...] = jnp.dot(p, x_ref[...], preferred_element_type=jnp.float32)


def _mm1_body(te_ref, tv_ref, xs_ref, w1_ref, b1_ref, h_ref):
    r = pl.program_id(1)

    @pl.when(tv_ref[r] == 1)
    def _():
        h_ref[...] = _gelu(
            jnp.dot(xs_ref[...], w1_ref[0], preferred_element_type=jnp.float32)
            + b1_ref[0])

    @pl.when(tv_ref[r] == 0)
    def _():
        h_ref[...] = jnp.zeros_like(h_ref)


def _mm2_body(te_ref, tv_ref, h_ref, w2_ref, b2_ref, y_ref):
    r = pl.program_id(0)

    @pl.when(tv_ref[r] == 1)
    def _():
        y_ref[...] = jax.nn.sigmoid(
            jnp.dot(h_ref[...], w2_ref[0], preferred_element_type=jnp.float32)
            + b2_ref[0])

    @pl.when(tv_ref[r] == 0)
    def _():
        y_ref[...] = jnp.zeros_like(y_ref)


def _combine_body(p0_ref, p1_ref, w0_ref, w1_ref, y_ref, out_ref):
    coln = jax.lax.broadcasted_iota(jnp.int32, (TB, N), 1)
    s = (jnp.where(coln == p0_ref[...], w0_ref[...], 0.0)
         + jnp.where(coln == p1_ref[...], w1_ref[...], 0.0))
    out_ref[...] = jnp.dot(s, y_ref[...], preferred_element_type=jnp.float32)


def kernel(combined, Wg1, bg1, Wg2, bg2, W1, b1, W2, b2):
    f32 = jnp.float32
    wg2p = jnp.pad(Wg2, ((0, 0), (0, EPAD - E)))
    bg2p = jnp.pad(bg2, (0, EPAD - E)).reshape(1, EPAD)
    bg1r = bg1.reshape(1, D)
    meta = pl.pallas_call(
        _gate_body,
        grid=(T // TB,),
        in_specs=[
            pl.BlockSpec((TB, DIN), lambda i: (i, 0)),
            pl.BlockSpec((DIN, D), lambda i: (0, 0)),
            pl.BlockSpec((1, D), lambda i: (0, 0)),
            pl.BlockSpec((D, EPAD), lambda i: (0, 0)),
            pl.BlockSpec((1, EPAD), lambda i: (0, 0)),
        ],
        out_specs=pl.BlockSpec((TB, 8), lambda i: (i, 0)),
        out_shape=jax.ShapeDtypeStruct((T, 8), f32),
    )(combined, Wg1, bg1r, wg2p, bg2p)

    e0 = meta[:, 0].astype(jnp.int32)
    e1 = meta[:, 1].astype(jnp.int32)
    w0 = meta[:, 2:3]
    w1v = meta[:, 3:4]

    e_all = jnp.stack([e0, e1], axis=1).reshape(-1)
    onehot = (e_all[:, None] == jnp.arange(E, dtype=jnp.int32)[None, :]).astype(jnp.int32)
    ranks = jnp.cumsum(onehot, axis=0)
    counts = ranks[-1]
    rank_p = jnp.take_along_axis(ranks, e_all[:, None], axis=1)[:, 0] - 1
    padded = ((counts + B - 1) // B) * B
    poff = jnp.concatenate([jnp.zeros(1, jnp.int32),
                            jnp.cumsum(padded)[:-1].astype(jnp.int32)])
    dst = poff[e_all] + rank_p
    tok = jnp.arange(T * K, dtype=jnp.int32) // K
    row_token = jnp.zeros(N, jnp.int32).at[dst].set(tok)
    pos = dst.reshape(T, K)
    tile_start = jnp.arange(R, dtype=jnp.int32) * B
    te = jnp.clip(jnp.searchsorted(poff, tile_start, side='right') - 1,
                  0, E - 1).astype(jnp.int32)
    tv = (tile_start < poff[te] + counts[te]).astype(jnp.int32)

    xs = pl.pallas_call(
        _gather_body,
        grid=(R,),
        in_specs=[
            pl.BlockSpec((1, B, 1), lambda r: (r, 0, 0)),
            pl.BlockSpec((T, DIN), lambda r: (0, 0)),
        ],
        out_specs=pl.BlockSpec((B, DIN), lambda r: (r, 0)),
        out_shape=jax.ShapeDtypeStruct((N, DIN), f32),
    )(row_token.reshape(R, B, 1), combined)

    b1r = b1.reshape(E * NF, 1, FB)
    h = pl.pallas_call(
        _mm1_body,
        grid_spec=pltpu.PrefetchScalarGridSpec(
            num_scalar_prefetch=2,
            grid=(NF, R),
            in_specs=[
                pl.BlockSpec((B, DIN), lambda f, r, te, tv: (r, 0)),
                pl.BlockSpec((1, DIN, FB), lambda f, r, te, tv: (te[r], 0, f)),
                pl.BlockSpec((1, 1, FB), lambda f, r, te, tv: (te[r] * NF + f, 0, 0)),
            ],
            out_specs=pl.BlockSpec((B, FB), lambda f, r, te, tv: (r, f)),
        ),
        out_shape=jax.ShapeDtypeStruct((N, F), f32),
    )(te, tv, xs, W1, b1r)

    b2r = b2.reshape(E, 1, D)
    y = pl.pallas_call(
        _mm2_body,
        grid_spec=pltpu.PrefetchScalarGridSpec(
            num_scalar_prefetch=2,
            grid=(R,),
            in_specs=[
                pl.BlockSpec((B, F), lambda r, te, tv: (r, 0)),
                pl.BlockSpec((1, F, D), lambda r, te, tv: (te[r], 0, 0)),
                pl.BlockSpec((1, 1, D), lambda r, te, tv: (te[r], 0, 0)),
            ],
            out_specs=pl.BlockSpec((B, D), lambda r, te, tv: (r, 0)),
        ),
        out_shape=jax.ShapeDtypeStruct((N, D), f32),
    )(te, tv, h, W2, b2r)

    fused = pl.pallas_call(
        _combine_body,
        grid=(T // TB,),
        in_specs=[
            pl.BlockSpec((TB, 1), lambda i: (i, 0)),
            pl.BlockSpec((TB, 1), lambda i: (i, 0)),
            pl.BlockSpec((TB, 1), lambda i: (i, 0)),
            pl.BlockSpec((TB, 1), lambda i: (i, 0)),
            pl.BlockSpec((N, D), lambda i: (0, 0)),
        ],
        out_specs=pl.BlockSpec((TB, D), lambda i: (i, 0)),
        out_shape=jax.ShapeDtypeStruct((T, D), f32),
    )(pos[:, 0:1], pos[:, 1:2], w0, w1v, y)
    return fused
